# baseline (speedup 1.0000x reference)
import functools

import jax
import jax.numpy as jnp
import numpy as np
from jax import lax
from jax.experimental import pallas as pl
from jax.experimental.pallas import tpu as pltpu
from jax.experimental.pallas import tpu_sc as plsc

_B, _N, _C = 32, 8192, 64
_NPOINTS = 2048
_NW = 32
_PER_W = (_B * _NPOINTS) // _NW
_CHUNK = 128
_NCHUNK = _PER_W // _CHUNK

_IDX_CONST = None


def _flat_indices() -> np.ndarray:
    global _IDX_CONST
    if _IDX_CONST is None:
        with jax.ensure_compile_time_eval():
            idx = jax.random.randint(
                jax.random.key(42), (_B, _NPOINTS), 0, _N, dtype=jnp.int32)
            flat = idx + jnp.arange(_B, dtype=jnp.int32)[:, None] * _N
            _IDX_CONST = np.asarray(flat).reshape(_NW * _NCHUNK, _CHUNK)
    return _IDX_CONST


def _sampler_body(table_hbm, idx_hbm, out_hbm, idx_v, rows_v, gsem):
    wid = lax.axis_index("s") * 2 + lax.axis_index("c")
    row_base = wid * _PER_W
    pltpu.sync_copy(idx_hbm.at[pl.ds(wid * _NCHUNK, _NCHUNK)], idx_v)
    for j in range(_NCHUNK):
        pltpu.async_copy(table_hbm.at[idx_v.at[j]], rows_v, gsem).wait()
        pltpu.sync_copy(rows_v, out_hbm.at[pl.ds(row_base + j * _CHUNK, _CHUNK)])


@functools.partial(jax.jit, static_argnames=())
def _sampler(table, idx2d):
    mesh = plsc.VectorSubcoreMesh(core_axis_name="c", subcore_axis_name="s")
    call = pl.kernel(
        _sampler_body,
        out_type=jax.ShapeDtypeStruct((_B * _NPOINTS, _C), jnp.float32),
        mesh=mesh,
        scratch_types=[
            pltpu.VMEM((_NCHUNK, _CHUNK), jnp.int32),
            pltpu.VMEM((_CHUNK, _C), jnp.float32),
            pltpu.SemaphoreType.DMA,
        ],
        compiler_params=pltpu.CompilerParams(use_tc_tiling_on_sc=False),
    )
    return call(table, idx2d)


def kernel(inputs):
    table = inputs.reshape(_B * _N, _C)
    idx2d = jnp.asarray(_flat_indices())
    out = _sampler(table, idx2d)
    return out.reshape(_B, _NPOINTS, _C)


# speedup vs baseline: 1.0438x; 1.0438x over previous
import functools

import jax
import jax.numpy as jnp
import numpy as np
from jax import lax
from jax.experimental import pallas as pl
from jax.experimental.pallas import tpu as pltpu
from jax.experimental.pallas import tpu_sc as plsc

_B, _N, _C = 32, 8192, 64
_NPOINTS = 2048
_NW = 32
_PER_W = (_B * _NPOINTS) // _NW
_CHUNK = 128
_NCHUNK = _PER_W // _CHUNK
_GCHUNKS = 4
_GROUP = _CHUNK * _GCHUNKS
_NGROUP = _PER_W // _GROUP

_IDX_CONST = None


def _flat_indices() -> np.ndarray:
    global _IDX_CONST
    if _IDX_CONST is None:
        with jax.ensure_compile_time_eval():
            idx = jax.random.randint(
                jax.random.key(42), (_B, _NPOINTS), 0, _N, dtype=jnp.int32)
            flat = idx + jnp.arange(_B, dtype=jnp.int32)[:, None] * _N
            _IDX_CONST = np.asarray(flat).reshape(_NW * _NCHUNK, _CHUNK)
    return _IDX_CONST


def _sampler_body(table_hbm, idx_hbm, out_hbm, idx_v, rows_v,
                  gsem0, gsem1, osem0, osem1):
    gsems, osems = (gsem0, gsem1), (osem0, osem1)
    wid = lax.axis_index("s") * 2 + lax.axis_index("c")
    row_base = wid * _PER_W
    pltpu.sync_copy(idx_hbm.at[pl.ds(wid * _NCHUNK, _NCHUNK)], idx_v)

    def start_group(g):
        ph = g % 2
        handles = []
        for c in range(_GCHUNKS):
            j = g * _GCHUNKS + c
            dst = rows_v.at[ph].at[pl.ds(c * _CHUNK, _CHUNK)]
            handles.append(
                pltpu.async_copy(table_hbm.at[idx_v.at[j]], dst, gsems[ph]))
        return handles

    gh = {0: start_group(0)}
    oh = {}
    for g in range(_NGROUP):
        ph = g % 2
        if g + 1 < _NGROUP:
            if g + 1 >= 2:
                oh[g - 1].wait()
            gh[g + 1] = start_group(g + 1)
        for h in gh[g]:
            h.wait()
        oh[g] = pltpu.async_copy(
            rows_v.at[ph],
            out_hbm.at[pl.ds(row_base + g * _GROUP, _GROUP)], osems[ph])
    oh[_NGROUP - 2].wait()
    oh[_NGROUP - 1].wait()


@functools.partial(jax.jit, static_argnames=())
def _sampler(table, idx2d):
    mesh = plsc.VectorSubcoreMesh(core_axis_name="c", subcore_axis_name="s")
    call = pl.kernel(
        _sampler_body,
        out_type=jax.ShapeDtypeStruct((_B * _NPOINTS, _C), jnp.float32),
        mesh=mesh,
        scratch_types=[
            pltpu.VMEM((_NCHUNK, _CHUNK), jnp.int32),
            pltpu.VMEM((2, _GROUP, _C), jnp.float32),
            pltpu.SemaphoreType.DMA,
            pltpu.SemaphoreType.DMA,
            pltpu.SemaphoreType.DMA,
            pltpu.SemaphoreType.DMA,
        ],
        compiler_params=pltpu.CompilerParams(use_tc_tiling_on_sc=False),
    )
    return call(table, idx2d)


def kernel(inputs):
    table = inputs.reshape(_B * _N, _C)
    idx2d = jnp.asarray(_flat_indices())
    out = _sampler(table, idx2d)
    return out.reshape(_B, _NPOINTS, _C)
